# baseline (speedup 1.0000x reference)
import jax
import jax.numpy as jnp
from jax import lax
from jax.experimental import pallas as pl
from jax.experimental.pallas import tpu as pltpu

_B, _L, _D = 4, 1024, 256
_C = 128
_NC = _L // _C
_BB = 4

_PREC = None


def _bdot_tt(a, b):
    return lax.dot_general(a, b, (((2,), (2,)), ((0,), (0,))),
                           preferred_element_type=jnp.float32,
                           precision=_PREC)


def _body(q_ref, k_ref, pprev_ref, gain_ref, oscale_ref,
          qout_ref, pfin_ref):
    c = pl.program_id(1)

    @pl.when(c == 0)
    def _():
        pfin_ref[...] = pprev_ref[...]

    P = pfin_ref[...]
    F_in = jnp.sum(P * P, axis=(1, 2), keepdims=True)
    Q = q_ref[...]
    K = k_ref[...]

    row = lax.broadcasted_iota(jnp.int32, (_C, _C), 0)
    col = lax.broadcasted_iota(jnp.int32, (_C, _C), 1)

    num_inter = _bdot_tt(Q, P)
    KP = _bdot_tt(K, P)
    d = jnp.sum(KP * K, axis=2, keepdims=True)

    S = _bdot_tt(Q, K)
    G = _bdot_tt(K, K)
    S_causal = jnp.where(col <= row, S, 0.0)
    num_intra = lax.dot_general(S_causal, K, (((2,), (1,)), ((0,), (0,))),
                                preferred_element_type=jnp.float32,
                                precision=_PREC)
    numer = num_inter + num_intra

    G2 = G * G
    W = jnp.where(col < row, 2.0, jnp.where(col == row, 1.0, 0.0))
    w_row = jnp.sum(G2 * W, axis=2, keepdims=True)

    tri = jnp.where(col <= row, jnp.float32(1.0), jnp.float32(0.0))
    tri_b = jnp.broadcast_to(tri, (_BB, _C, _C))
    cs = lax.dot_general(tri_b, 2.0 * d + w_row,
                         (((2,), (1,)), ((0,), (0,))),
                         preferred_element_type=jnp.float32,
                         precision=_PREC)
    fro = jnp.sqrt(F_in + cs)

    q_aligned = numer * (1.0 / (fro + 1e-7))
    gain = jnp.exp(gain_ref[...])
    qout_ref[...] = jnp.tanh(q_aligned * gain) * oscale_ref[...]

    pfin_ref[...] = P + lax.dot_general(K, K, (((1,), (1,)), ((0,), (0,))),
                                        preferred_element_type=jnp.float32,
                                        precision=_PREC)


@jax.jit
def kernel(q, k, P_prev, log_gain, output_scale):
    gain2d = log_gain.reshape(1, _D)
    oscale2d = output_scale.reshape(1, _D)
    q_out, P_final = pl.pallas_call(
        _body,
        out_shape=(
            jax.ShapeDtypeStruct((_B, _L, _D), jnp.float32),
            jax.ShapeDtypeStruct((_B, _D, _D), jnp.float32),
        ),
        grid=(_B // _BB, _NC),
        in_specs=[
            pl.BlockSpec((_BB, _C, _D), lambda b, c: (b, c, 0)),
            pl.BlockSpec((_BB, _C, _D), lambda b, c: (b, c, 0)),
            pl.BlockSpec((_BB, _D, _D), lambda b, c: (b, 0, 0)),
            pl.BlockSpec((1, _D), lambda b, c: (0, 0)),
            pl.BlockSpec((1, _D), lambda b, c: (0, 0)),
        ],
        out_specs=(
            pl.BlockSpec((_BB, _C, _D), lambda b, c: (b, c, 0)),
            pl.BlockSpec((_BB, _D, _D), lambda b, c: (b, 0, 0)),
        ),
        compiler_params=pltpu.CompilerParams(
            dimension_semantics=("parallel", "arbitrary"),
        ),
        name="qkproj_chunked",
    )(q, k, P_prev, gain2d, oscale2d)
    return q_out, P_final


# speedup vs baseline: 1.2265x; 1.2265x over previous
import jax
import jax.numpy as jnp
from jax import lax
from jax.experimental import pallas as pl
from jax.experimental.pallas import tpu as pltpu

_B, _L, _D = 4, 1024, 256
_C = 256
_NC = _L // _C
_BB = 4

_PREC = None


def _bdot_tt(a, b):
    return lax.dot_general(a, b, (((2,), (2,)), ((0,), (0,))),
                           preferred_element_type=jnp.float32,
                           precision=_PREC)


def _body(q_ref, k_ref, pprev_ref, gain_ref, oscale_ref,
          qout_ref, pfin_ref):
    c = pl.program_id(1)

    @pl.when(c == 0)
    def _():
        pfin_ref[...] = pprev_ref[...]

    P = pfin_ref[...]
    F_in = jnp.sum(P * P, axis=(1, 2), keepdims=True)
    Q = q_ref[...]
    K = k_ref[...]

    row = lax.broadcasted_iota(jnp.int32, (_C, _C), 0)
    col = lax.broadcasted_iota(jnp.int32, (_C, _C), 1)

    num_inter = _bdot_tt(Q, P)
    KP = _bdot_tt(K, P)
    d = jnp.sum(KP * K, axis=2, keepdims=True)

    S = _bdot_tt(Q, K)
    G = _bdot_tt(K, K)
    S_causal = jnp.where(col <= row, S, 0.0)
    num_intra = lax.dot_general(S_causal, K, (((2,), (1,)), ((0,), (0,))),
                                preferred_element_type=jnp.float32,
                                precision=_PREC)
    numer = num_inter + num_intra

    G2 = G * G
    W = jnp.where(col < row, 2.0, jnp.where(col == row, 1.0, 0.0))
    w_row = jnp.sum(G2 * W, axis=2, keepdims=True)

    tri = jnp.where(col <= row, jnp.float32(1.0), jnp.float32(0.0))
    tri_b = jnp.broadcast_to(tri, (_BB, _C, _C))
    cs = lax.dot_general(tri_b, 2.0 * d + w_row,
                         (((2,), (1,)), ((0,), (0,))),
                         preferred_element_type=jnp.float32,
                         precision=_PREC)
    fro = jnp.sqrt(F_in + cs)

    q_aligned = numer * (1.0 / (fro + 1e-7))
    gain = jnp.exp(gain_ref[...])
    qout_ref[...] = jnp.tanh(q_aligned * gain) * oscale_ref[...]

    pfin_ref[...] = P + lax.dot_general(K, K, (((1,), (1,)), ((0,), (0,))),
                                        preferred_element_type=jnp.float32,
                                        precision=_PREC)


@jax.jit
def kernel(q, k, P_prev, log_gain, output_scale):
    gain2d = log_gain.reshape(1, _D)
    oscale2d = output_scale.reshape(1, _D)
    q_out, P_final = pl.pallas_call(
        _body,
        out_shape=(
            jax.ShapeDtypeStruct((_B, _L, _D), jnp.float32),
            jax.ShapeDtypeStruct((_B, _D, _D), jnp.float32),
        ),
        grid=(_B // _BB, _NC),
        in_specs=[
            pl.BlockSpec((_BB, _C, _D), lambda b, c: (b, c, 0)),
            pl.BlockSpec((_BB, _C, _D), lambda b, c: (b, c, 0)),
            pl.BlockSpec((_BB, _D, _D), lambda b, c: (b, 0, 0)),
            pl.BlockSpec((1, _D), lambda b, c: (0, 0)),
            pl.BlockSpec((1, _D), lambda b, c: (0, 0)),
        ],
        out_specs=(
            pl.BlockSpec((_BB, _C, _D), lambda b, c: (b, c, 0)),
            pl.BlockSpec((_BB, _D, _D), lambda b, c: (b, 0, 0)),
        ),
        compiler_params=pltpu.CompilerParams(
            dimension_semantics=("parallel", "arbitrary"),
        ),
        name="qkproj_chunked",
    )(q, k, P_prev, gain2d, oscale2d)
    return q_out, P_final
